# baseline (speedup 1.0000x reference)
import functools

import jax
import jax.numpy as jnp
from jax.experimental import pallas as pl
from jax.experimental.pallas import tpu as pltpu


def _matmul_kernel(x_ref, w_ref, o_ref, *, cm):
    w = w_ref[...]
    for mi in range(x_ref.shape[0] // cm):
        sl = pl.ds(mi * cm, cm)
        o_ref[sl, :] = jnp.dot(
            x_ref[sl, :], w, preferred_element_type=jnp.float32
        )


def _pick_tn(n):
    return 512 if n >= 512 else n


def _pick_tm(m):
    for tm in (4096, 2048, 1024, 512, 256, 128, 64, 32, 16, 8):
        if m % tm == 0:
            return tm
    return m


def kernel(in_idx, tok_emb_table, pos_emb_table, w_out):
    b, s = in_idx.shape
    h = tok_emb_table.shape[1]
    v = w_out.shape[1]
    m = b * s

    x = (jnp.take(tok_emb_table, in_idx.reshape(-1), axis=0)
         + jnp.tile(pos_emb_table[:s], (b, 1))).astype(jnp.bfloat16)
    w_mx = w_out.astype(jnp.bfloat16)

    tn = _pick_tn(v)
    tm = _pick_tm(m)

    cm = 256 if tm % 256 == 0 else tm
    out2d = pl.pallas_call(
        functools.partial(_matmul_kernel, cm=cm),
        out_shape=jax.ShapeDtypeStruct((m, v), jnp.float32),
        grid=(m // tm, pl.cdiv(v, tn)),
        in_specs=[
            pl.BlockSpec((tm, h), lambda i, j: (i, 0)),
            pl.BlockSpec((h, tn), lambda i, j: (0, j)),
        ],
        out_specs=pl.BlockSpec((tm, tn), lambda i, j: (i, j)),
        compiler_params=pltpu.CompilerParams(
            dimension_semantics=("parallel", "arbitrary"),
        ),
    )(x, w_mx)

    return out2d.reshape(b, s, v)


# speedup vs baseline: 1.0272x; 1.0272x over previous
import functools

import jax
import jax.numpy as jnp
from jax.experimental import pallas as pl
from jax.experimental.pallas import tpu as pltpu


def _matmul_kernel(x_ref, w_ref, o_ref, *, cm):
    w = w_ref[...]
    for mi in range(x_ref.shape[0] // cm):
        sl = pl.ds(mi * cm, cm)
        o_ref[sl, :] = jnp.dot(
            x_ref[sl, :], w, preferred_element_type=jnp.float32
        )


def _pick_tn(n):
    return 1024 if n >= 1024 else n


def _pick_tm(m):
    for tm in (4096, 2048, 1024, 512, 256, 128, 64, 32, 16, 8):
        if m % tm == 0:
            return tm
    return m


def kernel(in_idx, tok_emb_table, pos_emb_table, w_out):
    b, s = in_idx.shape
    h = tok_emb_table.shape[1]
    v = w_out.shape[1]
    m = b * s

    x = (jnp.take(tok_emb_table, in_idx.reshape(-1), axis=0)
         + jnp.tile(pos_emb_table[:s], (b, 1))).astype(jnp.bfloat16)
    w_mx = w_out.astype(jnp.bfloat16)

    tn = _pick_tn(v)
    tm = _pick_tm(m)

    cm = 256 if tm % 256 == 0 else tm
    out2d = pl.pallas_call(
        functools.partial(_matmul_kernel, cm=cm),
        out_shape=jax.ShapeDtypeStruct((m, v), jnp.float32),
        grid=(m // tm, pl.cdiv(v, tn)),
        in_specs=[
            pl.BlockSpec((tm, h), lambda i, j: (i, 0)),
            pl.BlockSpec((h, tn), lambda i, j: (0, j)),
        ],
        out_specs=pl.BlockSpec((tm, tn), lambda i, j: (i, j)),
        compiler_params=pltpu.CompilerParams(
            dimension_semantics=("parallel", "arbitrary"),
        ),
    )(x, w_mx)

    return out2d.reshape(b, s, v)


# speedup vs baseline: 1.1171x; 1.0876x over previous
import functools

import jax
import jax.numpy as jnp
from jax.experimental import pallas as pl
from jax.experimental.pallas import tpu as pltpu


def _matmul_kernel(x_ref, w_ref, o_ref, *, cm):
    w = w_ref[...].astype(jnp.bfloat16)
    for mi in range(x_ref.shape[0] // cm):
        sl = pl.ds(mi * cm, cm)
        o_ref[sl, :] = jnp.dot(
            x_ref[sl, :], w, preferred_element_type=jnp.float32
        )


def _pick_tn(n):
    return 1024 if n >= 1024 else n


def _pick_tm(m):
    for tm in (4096, 2048, 1024, 512, 256, 128, 64, 32, 16, 8):
        if m % tm == 0:
            return tm
    return m


def kernel(in_idx, tok_emb_table, pos_emb_table, w_out):
    b, s = in_idx.shape
    h = tok_emb_table.shape[1]
    v = w_out.shape[1]
    m = b * s

    x = (jnp.take(tok_emb_table, in_idx.reshape(-1), axis=0)
         + jnp.tile(pos_emb_table[:s], (b, 1))).astype(jnp.bfloat16)

    tn = _pick_tn(v)
    tm = _pick_tm(m)

    cm = 256 if tm % 256 == 0 else tm
    out2d = pl.pallas_call(
        functools.partial(_matmul_kernel, cm=cm),
        out_shape=jax.ShapeDtypeStruct((m, v), jnp.float32),
        grid=(m // tm, pl.cdiv(v, tn)),
        in_specs=[
            pl.BlockSpec((tm, h), lambda i, j: (i, 0)),
            pl.BlockSpec((h, tn), lambda i, j: (0, j)),
        ],
        out_specs=pl.BlockSpec((tm, tn), lambda i, j: (i, j)),
        compiler_params=pltpu.CompilerParams(
            dimension_semantics=("parallel", "arbitrary"),
        ),
    )(x, w_out)

    return out2d.reshape(b, s, v)


# speedup vs baseline: 1.1326x; 1.0139x over previous
import functools

import jax
import jax.numpy as jnp
from jax.experimental import pallas as pl
from jax.experimental.pallas import tpu as pltpu


def _matmul_kernel(x_ref, w_ref, o_ref, *, cm):
    w = w_ref[...].astype(jnp.bfloat16)
    for mi in range(x_ref.shape[0] // cm):
        sl = pl.ds(mi * cm, cm)
        o_ref[sl, :] = jnp.dot(
            x_ref[sl, :], w, preferred_element_type=jnp.float32
        )


def _pick_tn(n):
    return 512 if n >= 512 else n


def _pick_tm(m):
    for tm in (8192, 4096, 2048, 1024, 512, 256, 128, 64, 32, 16, 8):
        if m % tm == 0:
            return tm
    return m


def kernel(in_idx, tok_emb_table, pos_emb_table, w_out):
    b, s = in_idx.shape
    h = tok_emb_table.shape[1]
    v = w_out.shape[1]
    m = b * s

    x = (jnp.take(tok_emb_table, in_idx.reshape(-1), axis=0)
         + jnp.tile(pos_emb_table[:s], (b, 1))).astype(jnp.bfloat16)

    tn = _pick_tn(v)
    tm = _pick_tm(m)

    cm = 256 if tm % 256 == 0 else tm
    out2d = pl.pallas_call(
        functools.partial(_matmul_kernel, cm=cm),
        out_shape=jax.ShapeDtypeStruct((m, v), jnp.float32),
        grid=(m // tm, pl.cdiv(v, tn)),
        in_specs=[
            pl.BlockSpec((tm, h), lambda i, j: (i, 0)),
            pl.BlockSpec((h, tn), lambda i, j: (0, j)),
        ],
        out_specs=pl.BlockSpec((tm, tn), lambda i, j: (i, j)),
        compiler_params=pltpu.CompilerParams(
            dimension_semantics=("parallel", "arbitrary"),
        ),
    )(x, w_out)

    return out2d.reshape(b, s, v)
